# baseline (speedup 1.0000x reference)
import functools

import jax
import jax.numpy as jnp
from jax import lax
from jax.experimental import pallas as pl
from jax.experimental.pallas import tpu as pltpu
from jax.experimental.pallas import tpu_sc as plsc

_L = 16
_NC = 2
_NS = 16
_NEG = -1e30


def _sc_attention(hid_1d, tok_pad, w2, B, S, H):
    BPC = B // _NC
    SPW = S // _NS
    NG = SPW // _L
    NP = NG // 2
    HC = H // _L
    HG = H // _NS
    DU = 8
    PU = 4
    GW = 2 * _L * H

    mesh = plsc.VectorSubcoreMesh(core_axis_name="c", subcore_axis_name="s")

    @functools.partial(
        pl.kernel,
        mesh=mesh,
        compiler_params=pltpu.CompilerParams(
            needs_layout_passes=False, use_tc_tiling_on_sc=False),
        out_type=[
            jax.ShapeDtypeStruct((B, S), jnp.float32),
            jax.ShapeDtypeStruct((B, H), jnp.float32),
        ],
        scratch_types=[
            pltpu.VMEM((GW,), jnp.float32),
            pltpu.VMEM((GW,), jnp.float32),
            pltpu.VMEM((BPC * H,), jnp.float32),
            pltpu.VMEM((H,), jnp.float32),
            pltpu.VMEM((BPC * SPW,), jnp.float32),
            pltpu.VMEM((SPW,), jnp.float32),
            pltpu.VMEM((_L,), jnp.float32),
            pltpu.VMEM((_L,), jnp.float32),
            pltpu.VMEM((BPC, _L), jnp.float32),
            pltpu.VMEM((BPC, _NS, _L), jnp.float32),
            pltpu.VMEM((_NS, HG), jnp.float32),
            pltpu.VMEM((HG,), jnp.float32),
            pltpu.VMEM((B, _L), jnp.int32),
            pltpu.VMEM_SHARED((BPC, _NS, _L), jnp.float32),
            pltpu.VMEM_SHARED((BPC, _NS, H), jnp.float32),
            pltpu.SemaphoreType.DMA,
            pltpu.SemaphoreType.DMA,
            pltpu.SemaphoreType.DMA,
        ],
    )
    def sc_k(hid_hbm, tok_hbm, w2_hbm, probs_hbm, pooled_hbm,
             gbufA, gbufB, acc, w2v, t_buf, p_buf, m_ref, d_ref, row_buf,
             mdloc, mergebuf, poolbuf, tokv, sh_md, sh_acc,
             semA, semB, semM):
        cid = lax.axis_index("c")
        wid = lax.axis_index("s")
        pltpu.sync_copy(w2_hbm, w2v)
        pltpu.sync_copy(tok_hbm, tokv)
        zero16 = jnp.zeros((_L,), jnp.float32)
        lanes = jnp.arange(_L, dtype=jnp.int32)

        def batch_body(b_i, _):
            b = cid * BPC + b_i
            tv = tokv[b]
            st0 = tv[0]; en0 = tv[1]
            st1 = tv[2]; en1 = tv[3]
            st2 = tv[4]; en2 = tv[5]

            def blk_ov(p):
                lo = wid * SPW + p * (2 * _L)
                hi = lo + 2 * _L
                return (((st0 < hi) & (en0 > lo))
                        | ((st1 < hi) & (en1 > lo))
                        | ((st2 < hi) & (en2 > lo)))

            def src(p):
                return hid_hbm.at[
                    pl.ds((b * S + wid * SPW + p * (2 * _L)) * H, GW)]

            def zbody(hc, _):
                for u in range(PU):
                    acc[pl.ds(b_i * H + hc * (_L * PU) + u * _L, _L)] = zero16
                return 0
            lax.fori_loop(0, HC // PU, zbody, 0)
            m_ref[...] = jnp.full((_L,), _NEG, jnp.float32)
            d_ref[...] = zero16

            def compute(p, half, buf):
                base = half * _L * H

                def dbody(j, accs):
                    out = list(accs)
                    for u in range(DU):
                        off = j * (_L * DU) + u * _L
                        wv = w2v[pl.ds(off, _L)]
                        for s in range(_L):
                            out[s] = out[s] + buf[
                                pl.ds(base + s * H + off, _L)] * wv
                    return tuple(out)
                accs = lax.fori_loop(0, HC // DU, dbody, (zero16,) * _L)
                t_vec = zero16
                for s in range(_L):
                    ts = jnp.sum(accs[s])
                    t_vec = jnp.where(lanes == s,
                                      jnp.full((_L,), ts, jnp.float32),
                                      t_vec)
                s_lo = wid * SPW + p * (2 * _L) + half * _L
                t_buf[pl.ds(b_i * SPW + p * (2 * _L) + half * _L, _L)] = t_vec

                posg = lanes + s_lo
                mvec = (((posg >= st0) & (posg < en0))
                        | ((posg >= st1) & (posg < en1))
                        | ((posg >= st2) & (posg < en2)))
                m_old = m_ref[...][0]
                d_old = d_ref[...][0]
                tm = jnp.where(mvec, t_vec, jnp.float32(_NEG))
                m_new = jnp.maximum(m_old, jnp.max(tm))
                e_vec = jnp.where(mvec, jnp.exp(t_vec - m_new), 0.0)
                scale = jnp.exp(jnp.full((_L,), m_old - m_new,
                                         jnp.float32))[0]
                m_ref[...] = jnp.full((_L,), m_new, jnp.float32)
                d_ref[...] = jnp.full(
                    (_L,), d_old * scale + jnp.sum(e_vec), jnp.float32)
                es = [e_vec[s] for s in range(_L)]

                def pbody(j, _):
                    for u in range(PU):
                        off = j * (_L * PU) + u * _L
                        sl = pl.ds(b_i * H + off, _L)
                        a = acc[sl] * scale
                        for s in range(_L):
                            a = a + es[s] * buf[pl.ds(base + s * H + off, _L)]
                        acc[sl] = a
                    return 0
                lax.fori_loop(0, HC // PU, pbody, 0)

            def do_block(p, buf):
                compute(p, 0, buf)
                compute(p, 1, buf)

            @pl.when(blk_ov(0))
            def _():
                pltpu.make_async_copy(src(0), gbufA, semA).start()

            def pair_body(i, _):
                p0 = 2 * i
                p1 = 2 * i + 1
                p2 = 2 * i + 2

                @pl.when(blk_ov(p1))
                def _():
                    pltpu.make_async_copy(src(p1), gbufB, semB).start()

                @pl.when(blk_ov(p0))
                def _():
                    pltpu.make_async_copy(src(p0), gbufA, semA).wait()
                    do_block(p0, gbufA)

                @pl.when((p2 < NP) & blk_ov(p2))
                def _():
                    pltpu.make_async_copy(src(p2), gbufA, semA).start()

                @pl.when(blk_ov(p1))
                def _():
                    pltpu.make_async_copy(src(p1), gbufB, semB).wait()
                    do_block(p1, gbufB)
                return 0
            lax.fori_loop(0, NP // 2, pair_body, 0)

            m_fin = m_ref[...][0]
            d_fin = d_ref[...][0]
            md_vec = jnp.where(lanes == 0, jnp.full((_L,), m_fin, jnp.float32),
                               jnp.where(lanes == 1,
                                         jnp.full((_L,), d_fin, jnp.float32),
                                         zero16))
            row_buf[b_i] = md_vec
            pltpu.sync_copy(row_buf.at[b_i], sh_md.at[b_i, wid])
            pltpu.sync_copy(acc.at[pl.ds(b_i * H, H)], sh_acc.at[b_i, wid])
            return 0

        lax.fori_loop(0, BPC, batch_body, 0)
        plsc.subcore_barrier()

        pltpu.sync_copy(sh_md, mdloc)
        for b_i in range(BPC):
            b = cid * BPC + b_i
            tv = tokv[b]
            st0 = tv[0]; en0 = tv[1]
            st1 = tv[2]; en1 = tv[3]
            st2 = tv[4]; en2 = tv[5]

            m_all = zero16
            d_all = zero16
            for wi in range(_NS):
                row = mdloc[b_i, wi]
                sel = lanes == wi
                m_all = jnp.where(sel, jnp.full((_L,), row[0], jnp.float32),
                                  m_all)
                d_all = jnp.where(sel, jnp.full((_L,), row[1], jnp.float32),
                                  d_all)
            M = jnp.max(m_all)
            Mv = jnp.full((_L,), M, jnp.float32)
            ecorr = jnp.exp(m_all - Mv)
            Dv = jnp.full((_L,), jnp.sum(d_all * ecorr), jnp.float32)
            invD = jnp.where(Dv > 0,
                             jnp.ones((_L,), jnp.float32)
                             / jnp.maximum(Dv, jnp.float32(1e-30)),
                             zero16)
            ecs = [ecorr[wi] for wi in range(_NS)]

            for wi in range(_NS):
                pltpu.make_async_copy(
                    sh_acc.at[b_i, wi, pl.ds(wid * HG, HG)],
                    mergebuf.at[wi], semM).start()
            for wi in range(_NS):
                pltpu.make_async_copy(
                    sh_acc.at[b_i, wi, pl.ds(wid * HG, HG)],
                    mergebuf.at[wi], semM).wait()
            for ch in range(HG // _L):
                sl = pl.ds(ch * _L, _L)
                v = zero16
                for wi in range(_NS):
                    v = v + ecs[wi] * mergebuf[wi, sl]
                poolbuf[sl] = v * invD
            pltpu.sync_copy(poolbuf, pooled_hbm.at[b, pl.ds(wid * HG, HG)])

            def prbody(g, _):
                t_vec = t_buf[pl.ds(b_i * SPW + g * _L, _L)]
                posg = lanes + (wid * SPW + g * _L)
                mvec = (((posg >= st0) & (posg < en0))
                        | ((posg >= st1) & (posg < en1))
                        | ((posg >= st2) & (posg < en2)))
                p = jnp.where(mvec, jnp.exp(t_vec - Mv) * invD, 0.0)
                p_buf[pl.ds(g * _L, _L)] = p
                return 0
            lax.fori_loop(0, NG, prbody, 0)
            pltpu.sync_copy(p_buf, probs_hbm.at[b, pl.ds(wid * SPW, SPW)])

    return sc_k(hid_1d, tok_pad, w2)


def _proj_body(pooled_ref, wout_ref, bout_ref, proj_ref):
    proj_ref[...] = jnp.tanh(
        jnp.dot(pooled_ref[...], wout_ref[...],
                preferred_element_type=jnp.float32) + bout_ref[...])


def kernel(hidden, token_idxs, pooled_entities, W_align, b_align, W_out, b_out):
    B, S, H = hidden.shape
    OUT = W_out.shape[1]
    F = token_idxs.shape[0]
    T = token_idxs.shape[2]
    del pooled_entities, b_align

    tok = token_idxs.reshape(F * B, T * 2).astype(jnp.int32)
    tok_pad = jnp.pad(tok, ((0, 0), (0, _L - T * 2)))
    w2 = W_align[H:, 0]
    hid_1d = hidden.reshape(B * S * H)

    probs, pooled = _sc_attention(hid_1d, tok_pad, w2, B, S, H)

    proj = pl.pallas_call(
        _proj_body,
        out_shape=jax.ShapeDtypeStruct((B, OUT), jnp.float32),
        compiler_params=pltpu.CompilerParams(
            vmem_limit_bytes=100 * 1024 * 1024,
        ),
    )(pooled, W_out, b_out.reshape(1, OUT))

    return proj, probs.reshape(1, B, S, 1)


# speedup vs baseline: 1.0838x; 1.0838x over previous
import functools

import jax
import jax.numpy as jnp
from jax import lax
from jax.experimental import pallas as pl
from jax.experimental.pallas import tpu as pltpu
from jax.experimental.pallas import tpu_sc as plsc

_L = 16
_NC = 2
_NS = 16
_NEG = -1e30


def _sc_attention(hid_1d, tok_pad, w2, B, S, H):
    BPC = B // _NC
    SPW = S // _NS
    NG = SPW // _L
    NP = NG
    HC = H // _L
    HG = H // _NS
    DU = 8
    PU = 4
    GW = _L * H

    mesh = plsc.VectorSubcoreMesh(core_axis_name="c", subcore_axis_name="s")

    @functools.partial(
        pl.kernel,
        mesh=mesh,
        compiler_params=pltpu.CompilerParams(
            needs_layout_passes=False, use_tc_tiling_on_sc=False),
        out_type=[
            jax.ShapeDtypeStruct((B, S), jnp.float32),
            jax.ShapeDtypeStruct((B, H), jnp.float32),
        ],
        scratch_types=[
            pltpu.VMEM((GW,), jnp.float32),
            pltpu.VMEM((GW,), jnp.float32),
            pltpu.VMEM((BPC * H,), jnp.float32),
            pltpu.VMEM((H,), jnp.float32),
            pltpu.VMEM((BPC * SPW,), jnp.float32),
            pltpu.VMEM((SPW,), jnp.float32),
            pltpu.VMEM((_L,), jnp.float32),
            pltpu.VMEM((_L,), jnp.float32),
            pltpu.VMEM((BPC, _L), jnp.float32),
            pltpu.VMEM((BPC, _NS, _L), jnp.float32),
            pltpu.VMEM((_NS, HG), jnp.float32),
            pltpu.VMEM((HG,), jnp.float32),
            pltpu.VMEM((B, _L), jnp.int32),
            pltpu.VMEM_SHARED((BPC, _NS, _L), jnp.float32),
            pltpu.VMEM_SHARED((BPC, _NS, H), jnp.float32),
            pltpu.SemaphoreType.DMA,
            pltpu.SemaphoreType.DMA,
            pltpu.SemaphoreType.DMA,
        ],
    )
    def sc_k(hid_hbm, tok_hbm, w2_hbm, probs_hbm, pooled_hbm,
             gbufA, gbufB, acc, w2v, t_buf, p_buf, m_ref, d_ref, row_buf,
             mdloc, mergebuf, poolbuf, tokv, sh_md, sh_acc,
             semA, semB, semM):
        cid = lax.axis_index("c")
        wid = lax.axis_index("s")
        pltpu.sync_copy(w2_hbm, w2v)
        pltpu.sync_copy(tok_hbm, tokv)
        zero16 = jnp.zeros((_L,), jnp.float32)
        lanes = jnp.arange(_L, dtype=jnp.int32)

        def batch_body(b_i, _):
            b = cid * BPC + b_i
            tv = tokv[b]
            st0 = tv[0]; en0 = tv[1]
            st1 = tv[2]; en1 = tv[3]
            st2 = tv[4]; en2 = tv[5]

            def src(p):
                return hid_hbm.at[
                    pl.ds((b * S + wid * SPW + p * _L) * H, GW)]

            def zbody(hc, _):
                for u in range(PU):
                    acc[pl.ds(b_i * H + hc * (_L * PU) + u * _L, _L)] = zero16
                return 0
            lax.fori_loop(0, HC // PU, zbody, 0)
            m_ref[...] = jnp.full((_L,), _NEG, jnp.float32)
            d_ref[...] = zero16

            def compute(p, buf):
                def dbody(j, accs):
                    out = list(accs)
                    for u in range(DU):
                        off = j * (_L * DU) + u * _L
                        wv = w2v[pl.ds(off, _L)]
                        for s in range(_L):
                            out[s] = out[s] + buf[
                                pl.ds(s * H + off, _L)] * wv
                    return tuple(out)
                accs = lax.fori_loop(0, HC // DU, dbody, (zero16,) * _L)
                t_vec = zero16
                for s in range(_L):
                    ts = jnp.sum(accs[s])
                    t_vec = jnp.where(lanes == s,
                                      jnp.full((_L,), ts, jnp.float32),
                                      t_vec)
                s_lo = wid * SPW + p * _L
                t_buf[pl.ds(b_i * SPW + p * _L, _L)] = t_vec

                posg = lanes + s_lo
                mvec = (((posg >= st0) & (posg < en0))
                        | ((posg >= st1) & (posg < en1))
                        | ((posg >= st2) & (posg < en2)))
                m_old = m_ref[...][0]
                d_old = d_ref[...][0]
                tm = jnp.where(mvec, t_vec, jnp.float32(_NEG))
                m_new = jnp.maximum(m_old, jnp.max(tm))
                e_vec = jnp.where(mvec, jnp.exp(t_vec - m_new), 0.0)
                scale = jnp.exp(jnp.full((_L,), m_old - m_new,
                                         jnp.float32))[0]
                m_ref[...] = jnp.full((_L,), m_new, jnp.float32)
                d_ref[...] = jnp.full(
                    (_L,), d_old * scale + jnp.sum(e_vec), jnp.float32)
                es = [e_vec[s] for s in range(_L)]

                def pbody(j, _):
                    for u in range(PU):
                        off = j * (_L * PU) + u * _L
                        sl = pl.ds(b_i * H + off, _L)
                        a = acc[sl] * scale
                        for s in range(_L):
                            a = a + es[s] * buf[pl.ds(s * H + off, _L)]
                        acc[sl] = a
                    return 0
                lax.fori_loop(0, HC // PU, pbody, 0)

            pltpu.make_async_copy(src(0), gbufA, semA).start()

            def pair_body(i, _):
                p0 = 2 * i
                p1 = 2 * i + 1
                p2 = 2 * i + 2
                pltpu.make_async_copy(src(p1), gbufB, semB).start()
                pltpu.make_async_copy(src(p0), gbufA, semA).wait()
                compute(p0, gbufA)

                @pl.when(p2 < NP)
                def _():
                    pltpu.make_async_copy(src(p2), gbufA, semA).start()
                pltpu.make_async_copy(src(p1), gbufB, semB).wait()
                compute(p1, gbufB)
                return 0
            lax.fori_loop(0, NP // 2, pair_body, 0)

            m_fin = m_ref[...][0]
            d_fin = d_ref[...][0]
            md_vec = jnp.where(lanes == 0, jnp.full((_L,), m_fin, jnp.float32),
                               jnp.where(lanes == 1,
                                         jnp.full((_L,), d_fin, jnp.float32),
                                         zero16))
            row_buf[b_i] = md_vec
            pltpu.sync_copy(row_buf.at[b_i], sh_md.at[b_i, wid])
            pltpu.sync_copy(acc.at[pl.ds(b_i * H, H)], sh_acc.at[b_i, wid])
            return 0

        lax.fori_loop(0, BPC, batch_body, 0)
        plsc.subcore_barrier()

        pltpu.sync_copy(sh_md, mdloc)
        for b_i in range(BPC):
            b = cid * BPC + b_i
            tv = tokv[b]
            st0 = tv[0]; en0 = tv[1]
            st1 = tv[2]; en1 = tv[3]
            st2 = tv[4]; en2 = tv[5]

            m_all = zero16
            d_all = zero16
            for wi in range(_NS):
                row = mdloc[b_i, wi]
                sel = lanes == wi
                m_all = jnp.where(sel, jnp.full((_L,), row[0], jnp.float32),
                                  m_all)
                d_all = jnp.where(sel, jnp.full((_L,), row[1], jnp.float32),
                                  d_all)
            M = jnp.max(m_all)
            Mv = jnp.full((_L,), M, jnp.float32)
            ecorr = jnp.exp(m_all - Mv)
            Dv = jnp.full((_L,), jnp.sum(d_all * ecorr), jnp.float32)
            invD = jnp.where(Dv > 0,
                             jnp.ones((_L,), jnp.float32)
                             / jnp.maximum(Dv, jnp.float32(1e-30)),
                             zero16)
            ecs = [ecorr[wi] for wi in range(_NS)]

            for wi in range(_NS):
                pltpu.make_async_copy(
                    sh_acc.at[b_i, wi, pl.ds(wid * HG, HG)],
                    mergebuf.at[wi], semM).start()
            for wi in range(_NS):
                pltpu.make_async_copy(
                    sh_acc.at[b_i, wi, pl.ds(wid * HG, HG)],
                    mergebuf.at[wi], semM).wait()
            for ch in range(HG // _L):
                sl = pl.ds(ch * _L, _L)
                v = zero16
                for wi in range(_NS):
                    v = v + ecs[wi] * mergebuf[wi, sl]
                poolbuf[sl] = v * invD
            pltpu.sync_copy(poolbuf, pooled_hbm.at[b, pl.ds(wid * HG, HG)])

            def prbody(g, _):
                t_vec = t_buf[pl.ds(b_i * SPW + g * _L, _L)]
                posg = lanes + (wid * SPW + g * _L)
                mvec = (((posg >= st0) & (posg < en0))
                        | ((posg >= st1) & (posg < en1))
                        | ((posg >= st2) & (posg < en2)))
                p = jnp.where(mvec, jnp.exp(t_vec - Mv) * invD, 0.0)
                p_buf[pl.ds(g * _L, _L)] = p
                return 0
            lax.fori_loop(0, NG, prbody, 0)
            pltpu.sync_copy(p_buf, probs_hbm.at[b, pl.ds(wid * SPW, SPW)])

    return sc_k(hid_1d, tok_pad, w2)


def _proj_body(pooled_ref, wout_ref, bout_ref, proj_ref):
    proj_ref[...] = jnp.tanh(
        jnp.dot(pooled_ref[...], wout_ref[...],
                preferred_element_type=jnp.float32) + bout_ref[...])


def kernel(hidden, token_idxs, pooled_entities, W_align, b_align, W_out, b_out):
    B, S, H = hidden.shape
    OUT = W_out.shape[1]
    F = token_idxs.shape[0]
    T = token_idxs.shape[2]
    del pooled_entities, b_align

    tok = token_idxs.reshape(F * B, T * 2).astype(jnp.int32)
    tok_pad = jnp.pad(tok, ((0, 0), (0, _L - T * 2)))
    w2 = W_align[H:, 0]
    hid_1d = hidden.reshape(B * S * H)

    probs, pooled = _sc_attention(hid_1d, tok_pad, w2, B, S, H)

    proj = pl.pallas_call(
        _proj_body,
        out_shape=jax.ShapeDtypeStruct((B, OUT), jnp.float32),
        compiler_params=pltpu.CompilerParams(
            vmem_limit_bytes=100 * 1024 * 1024,
        ),
    )(pooled, W_out, b_out.reshape(1, OUT))

    return proj, probs.reshape(1, B, S, 1)


# speedup vs baseline: 1.7482x; 1.6130x over previous
import jax
import jax.numpy as jnp
from jax import lax
from jax.experimental import pallas as pl
from jax.experimental.pallas import tpu as pltpu

_BLK = 128


def _body(sched_ref, nm_ref, tok_ref, hid_ref, w2_ref, wout_ref, bout_ref,
          attn_ref, proj_ref, sc_ref, md_ref, acc_ref):
    b = pl.program_id(0)
    i = pl.program_id(1)
    NB = pl.num_programs(1)
    S = sc_ref.shape[0]
    T = tok_ref.shape[1]
    neg = jnp.float32(-1e30)

    @pl.when(i == 0)
    def _():
        md_ref[0, 0] = neg
        md_ref[0, 1] = 0.0
        acc_ref[...] = jnp.zeros_like(acc_ref)

    @pl.when(i < nm_ref[b])
    def _():
        k = sched_ref[b, i]
        hid = hid_ref[0]
        sc = jnp.dot(hid, w2_ref[...],
                     preferred_element_type=jnp.float32)
        sc_ref[pl.ds(k * _BLK, _BLK), :] = sc
        pos = lax.broadcasted_iota(jnp.int32, (_BLK, 1), 0) + k * _BLK
        mask = jnp.zeros((_BLK, 1), jnp.bool_)
        for t in range(T):
            mask = mask | ((pos >= tok_ref[b, t, 0]) & (pos < tok_ref[b, t, 1]))
        scm = jnp.where(mask, sc, neg)
        m_old = md_ref[0, 0]
        m_new = jnp.maximum(m_old, jnp.max(scm))
        scale = jnp.exp(m_old - m_new)
        e = jnp.where(mask, jnp.exp(sc - m_new), 0.0)
        md_ref[0, 0] = m_new
        md_ref[0, 1] = md_ref[0, 1] * scale + jnp.sum(e)
        acc_ref[...] = acc_ref[...] * scale + lax.dot_general(
            e, hid, (((0,), (0,)), ((), ())),
            preferred_element_type=jnp.float32)

    @pl.when(i == NB - 1)
    def _():
        m = md_ref[0, 0]
        d = md_ref[0, 1]
        scs = sc_ref[...]
        pos = lax.broadcasted_iota(jnp.int32, (S, 1), 0)
        mask = jnp.zeros((S, 1), jnp.bool_)
        for t in range(T):
            mask = mask | ((pos >= tok_ref[b, t, 0]) & (pos < tok_ref[b, t, 1]))
        e = jnp.where(mask, jnp.exp(scs - m), 0.0)
        inv = jnp.where(d > 0, 1.0 / jnp.maximum(d, jnp.float32(1e-30)), 0.0)
        attn_ref[0] = e * inv
        pooled = acc_ref[...] * inv
        proj = jnp.tanh(jnp.dot(pooled, wout_ref[...],
                                preferred_element_type=jnp.float32)
                        + bout_ref[...])
        proj_ref[pl.ds(b, 1), :] = proj


def kernel(hidden, token_idxs, pooled_entities, W_align, b_align, W_out, b_out):
    B, S, H = hidden.shape
    OUT = W_out.shape[1]
    F = token_idxs.shape[0]
    T = token_idxs.shape[2]
    del pooled_entities, b_align
    NB = S // _BLK

    tok = token_idxs.reshape(F * B, T, 2).astype(jnp.int32)
    w2 = W_align[H:, :]
    bout = b_out.reshape(1, OUT)

    k = jnp.arange(NB, dtype=jnp.int32)
    starts = tok[..., 0][:, :, None]
    ends = tok[..., 1][:, :, None]
    ov = jnp.any((starts < (k[None, None, :] + 1) * _BLK)
                 & (ends > k[None, None, :] * _BLK), axis=1)
    nm = jnp.sum(ov, axis=1).astype(jnp.int32)
    order = jnp.where(ov, k[None, :], NB + k[None, :])
    sched = jnp.argsort(order, axis=1).astype(jnp.int32)
    last = jnp.take_along_axis(sched, jnp.maximum(nm - 1, 0)[:, None], axis=1)
    ii = k[None, :]
    sched = jnp.where(ii < nm[:, None], sched, last)

    grid_spec = pltpu.PrefetchScalarGridSpec(
        num_scalar_prefetch=3,
        grid=(B, NB),
        in_specs=[
            pl.BlockSpec((1, _BLK, H), lambda b, i, sr, nr, tr: (b, sr[b, i], 0)),
            pl.BlockSpec((H, 1), lambda b, i, sr, nr, tr: (0, 0)),
            pl.BlockSpec((H, OUT), lambda b, i, sr, nr, tr: (0, 0)),
            pl.BlockSpec((1, OUT), lambda b, i, sr, nr, tr: (0, 0)),
        ],
        out_specs=[
            pl.BlockSpec((1, S, 1), lambda b, i, sr, nr, tr: (b, 0, 0)),
            pl.BlockSpec((B, OUT), lambda b, i, sr, nr, tr: (0, 0)),
        ],
        scratch_shapes=[
            pltpu.VMEM((S, 1), jnp.float32),
            pltpu.SMEM((1, 2), jnp.float32),
            pltpu.VMEM((1, H), jnp.float32),
        ],
    )
    attn, proj = pl.pallas_call(
        _body,
        grid_spec=grid_spec,
        out_shape=[
            jax.ShapeDtypeStruct((B, S, 1), jnp.float32),
            jax.ShapeDtypeStruct((B, OUT), jnp.float32),
        ],
        compiler_params=pltpu.CompilerParams(
            dimension_semantics=("arbitrary", "arbitrary"),
            vmem_limit_bytes=100 * 1024 * 1024,
        ),
    )(sched, nm, tok, hidden, w2, W_out, bout)

    return proj, attn[None]


# speedup vs baseline: 3.5922x; 2.0548x over previous
import jax
import jax.numpy as jnp
from jax.experimental import pallas as pl
from jax.experimental.pallas import tpu as pltpu


def _attn_body(tok_ref, hid_ref, w2_ref, wout_ref, bout_ref, attn_ref, proj_ref):
    b = pl.program_id(0)
    S = hid_ref.shape[1]
    hid = hid_ref[0]
    w2 = w2_ref[...]
    sc = jnp.dot(hid, w2, preferred_element_type=jnp.float32)

    pos = jax.lax.broadcasted_iota(jnp.int32, (S, 1), 0)
    mask = jnp.zeros((S, 1), jnp.bool_)
    for t in range(tok_ref.shape[1]):
        st = tok_ref[b, t, 0]
        en = tok_ref[b, t, 1]
        mask = mask | ((pos >= st) & (pos < en))

    neg = jnp.float32(-1e30)
    scm = jnp.where(mask, sc, neg)
    m = jnp.max(scm, axis=0, keepdims=True)
    m = jnp.where(m > neg * 0.5, m, 0.0)
    e = jnp.where(mask, jnp.exp(sc - m), 0.0)
    denom = jnp.sum(e, axis=0, keepdims=True)
    probs = jnp.where(denom > 0, e / jnp.maximum(denom, 1e-30), 0.0)
    attn_ref[0] = probs

    pooled = jax.lax.dot_general(probs, hid, (((0,), (0,)), ((), ())),
                                 preferred_element_type=jnp.float32)
    proj = jnp.tanh(jnp.dot(pooled, wout_ref[...],
                            preferred_element_type=jnp.float32) + bout_ref[...])
    proj_ref[pl.ds(b, 1), :] = proj


def kernel(hidden, token_idxs, pooled_entities, W_align, b_align, W_out, b_out):
    B, S, H = hidden.shape
    OUT = W_out.shape[1]
    F = token_idxs.shape[0]
    T = token_idxs.shape[2]
    del pooled_entities, b_align

    tok = token_idxs.reshape(F * B, T, 2).astype(jnp.int32)
    w2 = W_align[H:, :]
    bout = b_out.reshape(1, OUT)

    attn, proj = pl.pallas_call(
        _attn_body,
        grid=(B,),
        in_specs=[
            pl.BlockSpec(memory_space=pltpu.SMEM),
            pl.BlockSpec((1, S, H), lambda b: (b, 0, 0)),
            pl.BlockSpec((H, 1), lambda b: (0, 0)),
            pl.BlockSpec((H, OUT), lambda b: (0, 0)),
            pl.BlockSpec((1, OUT), lambda b: (0, 0)),
        ],
        out_specs=[
            pl.BlockSpec((1, S, 1), lambda b: (b, 0, 0)),
            pl.BlockSpec((B, OUT), lambda b: (0, 0)),
        ],
        out_shape=[
            jax.ShapeDtypeStruct((B, S, 1), jnp.float32),
            jax.ShapeDtypeStruct((B, OUT), jnp.float32),
        ],
        compiler_params=pltpu.CompilerParams(
            dimension_semantics=("arbitrary",),
            vmem_limit_bytes=100 * 1024 * 1024,
        ),
    )(tok, hidden, w2, W_out, bout)

    return proj, attn[None]


# speedup vs baseline: 4.1629x; 1.1589x over previous
import jax
import jax.numpy as jnp
from jax.experimental import pallas as pl
from jax.experimental.pallas import tpu as pltpu


def _attn_body(tok_ref, hid_ref, w2_ref, wout_ref, bout_ref, attn_ref, proj_ref):
    b = pl.program_id(0)
    S = hid_ref.shape[1]
    hid = hid_ref[0]
    w2 = w2_ref[...]
    sc = jnp.dot(hid, w2, preferred_element_type=jnp.float32)

    pos = jax.lax.broadcasted_iota(jnp.int32, (S, 1), 0)
    mask = jnp.zeros((S, 1), jnp.bool_)
    for t in range(tok_ref.shape[1]):
        st = tok_ref[b, t, 0]
        en = tok_ref[b, t, 1]
        mask = mask | ((pos >= st) & (pos < en))

    neg = jnp.float32(-1e30)
    scm = jnp.where(mask, sc, neg)
    m = jnp.max(scm, axis=0, keepdims=True)
    m = jnp.where(m > neg * 0.5, m, 0.0)
    e = jnp.where(mask, jnp.exp(sc - m), 0.0)
    denom = jnp.sum(e, axis=0, keepdims=True)
    probs = jnp.where(denom > 0, e / jnp.maximum(denom, 1e-30), 0.0)
    attn_ref[0] = probs.reshape(attn_ref.shape[1], attn_ref.shape[2])

    pooled = jax.lax.dot_general(probs, hid, (((0,), (0,)), ((), ())),
                                 preferred_element_type=jnp.float32)
    proj = jnp.tanh(jnp.dot(pooled, wout_ref[...],
                            preferred_element_type=jnp.float32) + bout_ref[...])
    proj_ref[pl.ds(b, 1), :] = proj


def kernel(hidden, token_idxs, pooled_entities, W_align, b_align, W_out, b_out):
    B, S, H = hidden.shape
    OUT = W_out.shape[1]
    F = token_idxs.shape[0]
    T = token_idxs.shape[2]
    del pooled_entities, b_align

    tok = token_idxs.reshape(F * B, T, 2).astype(jnp.int32)
    w2 = W_align[H:, :]
    bout = b_out.reshape(1, OUT)

    attn, proj = pl.pallas_call(
        _attn_body,
        grid=(B,),
        in_specs=[
            pl.BlockSpec(memory_space=pltpu.SMEM),
            pl.BlockSpec((1, S, H), lambda b: (b, 0, 0)),
            pl.BlockSpec((H, 1), lambda b: (0, 0)),
            pl.BlockSpec((H, OUT), lambda b: (0, 0)),
            pl.BlockSpec((1, OUT), lambda b: (0, 0)),
        ],
        out_specs=[
            pl.BlockSpec((1, S // 128, 128), lambda b: (b, 0, 0)),
            pl.BlockSpec((B, OUT), lambda b: (0, 0)),
        ],
        out_shape=[
            jax.ShapeDtypeStruct((B, S // 128, 128), jnp.float32),
            jax.ShapeDtypeStruct((B, OUT), jnp.float32),
        ],
        compiler_params=pltpu.CompilerParams(
            dimension_semantics=("arbitrary",),
            vmem_limit_bytes=100 * 1024 * 1024,
        ),
    )(tok, hidden, w2, W_out, bout)

    return proj, attn.reshape(1, B, S, 1)
